# baseline (speedup 1.0000x reference)
import jax
import jax.numpy as jnp
from jax import lax
from jax.experimental import pallas as pl
from jax.experimental.pallas import tpu as pltpu
from jax.experimental.pallas import tpu_sc as plsc

_D = 32
_BB = 128
_NW = 32
_KS = 2


def _sc_body(mel_tok, chd_tok, mel_emb, chd_emb, mel_pos, chd_pos,
             mel_z, chd_z, idx_all, pos_v, buf_v0, buf_v1, out_v0, out_v1,
             gsem0, gsem1, osem0, osem1):
    cid = lax.axis_index("c")
    sid = lax.axis_index("s")
    wid = sid * 2 + cid
    seq = mel_pos.shape[0]
    n_blocks = seq // _KS
    bufs = (buf_v0, buf_v1)
    outs = (out_v0, out_v1)
    gsems = (gsem0, gsem1)
    osems = (osem0, osem1)
    iota16 = lax.iota(jnp.int32, 16)
    rows16 = [iota16 + (g * 16) for g in range(_BB // 16)]

    def run_table(tok_hbm, table_hbm, pos_hbm, z_hbm):
        pltpu.sync_copy(tok_hbm.at[wid], idx_all)
        pltpu.sync_copy(pos_hbm, pos_v)

        def gather(b, half):
            pltpu.async_copy(
                table_hbm.at[idx_all.at[pl.ds(b * (_KS * _BB), _KS * _BB)]],
                bufs[half], gsems[half])

        def drain_out(half):
            pltpu.make_async_copy(outs[half],
                                  z_hbm.at[pl.ds(0, _KS), :, wid],
                                  osems[half]).wait()

        gather(0, 0)

        def block_pair(b2, carry):
            for half in range(2):
                b = 2 * b2 + half
                nxt = b + 1

                @pl.when(nxt < n_blocks)
                def _():
                    gather(nxt, 1 - half)

                pltpu.make_async_copy(
                    table_hbm.at[idx_all.at[pl.ds(b * (_KS * _BB),
                                                  _KS * _BB)]],
                    bufs[half], gsems[half]).wait()

                @pl.when(b >= 2)
                def _():
                    drain_out(half)

                for j in range(_KS):
                    s = b * _KS + j
                    jbuf = bufs[half].at[pl.ds(j * _BB, _BB)]
                    sfull = jnp.full((16,), s, jnp.int32)
                    for dt in range(_D // 8):
                        for di in range(8):
                            d = dt * 8 + di
                            cols = jnp.full((16,), d, jnp.int32)
                            p = plsc.load_gather(pos_v, [sfull, cols])
                            vs = [plsc.load_gather(jbuf, [rows16[g], cols])
                                  for g in range(_BB // 16)]
                            for g in range(_BB // 16):
                                outs[half][j, dt, di, pl.ds(g * 16, 16)] = (
                                    vs[g] + p)
                pltpu.async_copy(outs[half],
                                 z_hbm.at[pl.ds(b * _KS, _KS), :, wid],
                                 osems[half])
            return carry

        lax.fori_loop(0, n_blocks // 2, block_pair, 0)
        drain_out(0)
        drain_out(1)

    run_table(mel_tok, mel_emb, mel_pos, mel_z)
    run_table(chd_tok, chd_emb, chd_pos, chd_z)


def kernel(melody_tokens, chord_tokens, melody_emb, chord_emb, enc_pos, dec_pos):
    batch, seq = melody_tokens.shape
    nb = batch // _BB

    def to_worker_major(tok):
        t = jnp.transpose(tok.astype(jnp.int32))
        return t.reshape(seq, nb, _BB).transpose(1, 0, 2).reshape(nb, seq * _BB)

    mel_t = to_worker_major(melody_tokens)
    chd_t = to_worker_major(chord_tokens)

    mesh = plsc.VectorSubcoreMesh(
        core_axis_name="c", subcore_axis_name="s", num_cores=2, num_subcores=16
    )
    z_shape = (seq, _D // 8, batch // _BB, 8, _BB)
    run = pl.kernel(
        _sc_body,
        out_type=(
            jax.ShapeDtypeStruct(z_shape, jnp.float32),
            jax.ShapeDtypeStruct(z_shape, jnp.float32),
        ),
        mesh=mesh,
        scratch_types=[
            pltpu.VMEM((seq * _BB,), jnp.int32),
            pltpu.VMEM((seq, _D), jnp.float32),
            pltpu.VMEM((_KS * _BB, _D), jnp.float32),
            pltpu.VMEM((_KS * _BB, _D), jnp.float32),
            pltpu.VMEM((_KS, _D // 8, 8, _BB), jnp.float32),
            pltpu.VMEM((_KS, _D // 8, 8, _BB), jnp.float32),
            pltpu.SemaphoreType.DMA,
            pltpu.SemaphoreType.DMA,
            pltpu.SemaphoreType.DMA,
            pltpu.SemaphoreType.DMA,
        ],
        compiler_params=pltpu.CompilerParams(
            use_tc_tiling_on_sc=False, needs_layout_passes=False),
    )
    mel_z, chd_z = run(mel_t, chd_t, melody_emb, chord_emb,
                       enc_pos[:seq], dec_pos[:seq])
    mel_out = mel_z.transpose(2, 4, 0, 1, 3).reshape(batch, seq, _D)
    chd_out = chd_z.transpose(2, 4, 0, 1, 3).reshape(batch, seq, _D)
    return (mel_out, chd_out)


# speedup vs baseline: 1.0944x; 1.0944x over previous
import jax
import jax.numpy as jnp
from jax import lax
from jax.experimental import pallas as pl
from jax.experimental.pallas import tpu as pltpu
from jax.experimental.pallas import tpu_sc as plsc

_D = 32
_BB = 128
_NW = 32
_KS = 4


def _sc_body(mel_tok, chd_tok, mel_emb, chd_emb, mel_pos, chd_pos,
             mel_z, chd_z, idx_all, pos_v, buf_v0, buf_v1, out_v0, out_v1,
             gsem0, gsem1, osem0, osem1):
    cid = lax.axis_index("c")
    sid = lax.axis_index("s")
    wid = sid * 2 + cid
    seq = mel_pos.shape[0]
    n_blocks = seq // _KS
    bufs = (buf_v0, buf_v1)
    outs = (out_v0, out_v1)
    gsems = (gsem0, gsem1)
    osems = (osem0, osem1)
    iota16 = lax.iota(jnp.int32, 16)
    rows16 = [iota16 + (g * 16) for g in range(_BB // 16)]

    def run_table(tok_hbm, table_hbm, pos_hbm, z_hbm):
        pltpu.sync_copy(tok_hbm.at[wid], idx_all)
        pltpu.sync_copy(pos_hbm, pos_v)

        def gather(b, half):
            pltpu.async_copy(
                table_hbm.at[idx_all.at[pl.ds(b * (_KS * _BB), _KS * _BB)]],
                bufs[half], gsems[half])

        def drain_out(half):
            pltpu.make_async_copy(outs[half],
                                  z_hbm.at[pl.ds(0, _KS), :, wid],
                                  osems[half]).wait()

        gather(0, 0)

        def block_pair(b2, carry):
            for half in range(2):
                b = 2 * b2 + half
                nxt = b + 1

                @pl.when(nxt < n_blocks)
                def _():
                    gather(nxt, 1 - half)

                pltpu.make_async_copy(
                    table_hbm.at[idx_all.at[pl.ds(b * (_KS * _BB),
                                                  _KS * _BB)]],
                    bufs[half], gsems[half]).wait()

                @pl.when(b >= 2)
                def _():
                    drain_out(half)

                for j in range(_KS):
                    s = b * _KS + j
                    jbuf = bufs[half].at[pl.ds(j * _BB, _BB)]
                    sfull = jnp.full((16,), s, jnp.int32)
                    for dt in range(_D // 8):

                        def di_body(di, c3, j=j, jbuf=jbuf, sfull=sfull,
                                    dt=dt):
                            d = dt * 8 + di
                            cols = jnp.full((16,), d, jnp.int32)
                            p = plsc.load_gather(pos_v, [sfull, cols])
                            vs = [plsc.load_gather(jbuf, [rows16[g], cols])
                                  for g in range(_BB // 16)]
                            for g in range(_BB // 16):
                                outs[half][j, dt, di, pl.ds(g * 16, 16)] = (
                                    vs[g] + p)
                            return c3

                        lax.fori_loop(0, 8, di_body, 0)
                pltpu.async_copy(outs[half],
                                 z_hbm.at[pl.ds(b * _KS, _KS), :, wid],
                                 osems[half])
            return carry

        lax.fori_loop(0, n_blocks // 2, block_pair, 0)
        drain_out(0)
        drain_out(1)

    run_table(mel_tok, mel_emb, mel_pos, mel_z)
    run_table(chd_tok, chd_emb, chd_pos, chd_z)


def kernel(melody_tokens, chord_tokens, melody_emb, chord_emb, enc_pos, dec_pos):
    batch, seq = melody_tokens.shape
    nb = batch // _BB

    def to_worker_major(tok):
        t = jnp.transpose(tok.astype(jnp.int32))
        return t.reshape(seq, nb, _BB).transpose(1, 0, 2).reshape(nb, seq * _BB)

    mel_t = to_worker_major(melody_tokens)
    chd_t = to_worker_major(chord_tokens)

    mesh = plsc.VectorSubcoreMesh(
        core_axis_name="c", subcore_axis_name="s", num_cores=2, num_subcores=16
    )
    z_shape = (seq, _D // 8, batch // _BB, 8, _BB)
    run = pl.kernel(
        _sc_body,
        out_type=(
            jax.ShapeDtypeStruct(z_shape, jnp.float32),
            jax.ShapeDtypeStruct(z_shape, jnp.float32),
        ),
        mesh=mesh,
        scratch_types=[
            pltpu.VMEM((seq * _BB,), jnp.int32),
            pltpu.VMEM((seq, _D), jnp.float32),
            pltpu.VMEM((_KS * _BB, _D), jnp.float32),
            pltpu.VMEM((_KS * _BB, _D), jnp.float32),
            pltpu.VMEM((_KS, _D // 8, 8, _BB), jnp.float32),
            pltpu.VMEM((_KS, _D // 8, 8, _BB), jnp.float32),
            pltpu.SemaphoreType.DMA,
            pltpu.SemaphoreType.DMA,
            pltpu.SemaphoreType.DMA,
            pltpu.SemaphoreType.DMA,
        ],
        compiler_params=pltpu.CompilerParams(
            use_tc_tiling_on_sc=False, needs_layout_passes=False),
    )
    mel_z, chd_z = run(mel_t, chd_t, melody_emb, chord_emb,
                       enc_pos[:seq], dec_pos[:seq])
    mel_out = mel_z.transpose(2, 4, 0, 1, 3).reshape(batch, seq, _D)
    chd_out = chd_z.transpose(2, 4, 0, 1, 3).reshape(batch, seq, _D)
    return (mel_out, chd_out)


# speedup vs baseline: 3.9790x; 3.6358x over previous
import jax
import jax.numpy as jnp
from jax import lax
from jax.experimental import pallas as pl
from jax.experimental.pallas import tpu as pltpu
from jax.experimental.pallas import tpu_sc as plsc

_D = 32
_BB = 128
_NW = 32
_KS = 4


def _sc_body(mel_tok, chd_tok, mel_emb, chd_emb, mel_pos, chd_pos,
             mel_z, chd_z, idx_all, pb_v0, pb_v1, buf_v0, buf_v1,
             out_v0, out_v1, gsem0, gsem1, osem0, osem1):
    cid = lax.axis_index("c")
    sid = lax.axis_index("s")
    wid = sid * 2 + cid
    seq = mel_pos.shape[0]
    n_blocks = seq // _KS
    bufs = (buf_v0, buf_v1)
    pbufs = (pb_v0, pb_v1)
    outs = (out_v0, out_v1)
    gsems = (gsem0, gsem1)
    osems = (osem0, osem1)
    iota16 = lax.iota(jnp.int32, 16)
    rows16 = [iota16 + (g * 16) for g in range(_BB // 16)]

    def run_table(tok_hbm, table_hbm, pos_hbm, z_hbm):
        pltpu.sync_copy(tok_hbm.at[wid], idx_all)

        def gather(b, half):
            pltpu.async_copy(
                table_hbm.at[idx_all.at[pl.ds(b * (_KS * _BB), _KS * _BB)]],
                bufs[half], gsems[half])
            pltpu.async_copy(pos_hbm.at[pl.ds(b * _KS, _KS)], pbufs[half],
                             gsems[half])

        def drain_out(half):
            pltpu.make_async_copy(outs[half],
                                  z_hbm.at[pl.ds(0, _KS), :, wid],
                                  osems[half]).wait()

        gather(0, 0)

        def block_pair(b2, carry):
            for half in range(2):
                b = 2 * b2 + half
                nxt = b + 1

                @pl.when(nxt < n_blocks)
                def _():
                    gather(nxt, 1 - half)

                pltpu.make_async_copy(
                    table_hbm.at[idx_all.at[pl.ds(b * (_KS * _BB),
                                                  _KS * _BB)]],
                    bufs[half], gsems[half]).wait()
                pltpu.make_async_copy(pos_hbm.at[pl.ds(b * _KS, _KS)],
                                      pbufs[half], gsems[half]).wait()

                @pl.when(b >= 2)
                def _():
                    drain_out(half)

                for j in range(_KS):
                    jbuf = bufs[half].at[pl.ds(j * _BB, _BB)]
                    jfull = jnp.full((16,), j, jnp.int32)

                    def d_body(d, c3, jbuf=jbuf, jfull=jfull, half=half):
                        dvec = (iota16 + d) & (_D - 1)
                        dt_vec = dvec >> 3
                        di_vec = dvec & 7
                        p = plsc.load_gather(pbufs[half],
                                             [jfull, dvec, iota16])
                        vs = [plsc.load_gather(jbuf, [rows16[g], dvec])
                              for g in range(_BB // 16)]
                        for g in range(_BB // 16):
                            plsc.store_scatter(
                                outs[half],
                                [jfull, dt_vec, di_vec, rows16[g]],
                                vs[g] + p)
                        return c3

                    lax.fori_loop(0, _D, d_body, 0)
                pltpu.async_copy(outs[half],
                                 z_hbm.at[pl.ds(b * _KS, _KS), :, wid],
                                 osems[half])
            return carry

        lax.fori_loop(0, n_blocks // 2, block_pair, 0)
        drain_out(0)
        drain_out(1)

    run_table(mel_tok, mel_emb, mel_pos, mel_z)
    run_table(chd_tok, chd_emb, chd_pos, chd_z)


def kernel(melody_tokens, chord_tokens, melody_emb, chord_emb, enc_pos, dec_pos):
    batch, seq = melody_tokens.shape
    nb = batch // _BB

    def to_worker_major(tok):
        t = jnp.transpose(tok.astype(jnp.int32))
        return t.reshape(seq, nb, _BB).transpose(1, 0, 2).reshape(nb, seq * _BB)

    mel_t = to_worker_major(melody_tokens)
    chd_t = to_worker_major(chord_tokens)

    mesh = plsc.VectorSubcoreMesh(
        core_axis_name="c", subcore_axis_name="s", num_cores=2, num_subcores=16
    )
    z_shape = (seq, _D // 8, batch // _BB, 8, _BB)
    run = pl.kernel(
        _sc_body,
        out_type=(
            jax.ShapeDtypeStruct(z_shape, jnp.float32),
            jax.ShapeDtypeStruct(z_shape, jnp.float32),
        ),
        mesh=mesh,
        scratch_types=[
            pltpu.VMEM((seq * _BB,), jnp.int32),
            pltpu.VMEM((_KS, _D, 16), jnp.float32),
            pltpu.VMEM((_KS, _D, 16), jnp.float32),
            pltpu.VMEM((_KS * _BB, _D), jnp.float32),
            pltpu.VMEM((_KS * _BB, _D), jnp.float32),
            pltpu.VMEM((_KS, _D // 8, 8, _BB), jnp.float32),
            pltpu.VMEM((_KS, _D // 8, 8, _BB), jnp.float32),
            pltpu.SemaphoreType.DMA,
            pltpu.SemaphoreType.DMA,
            pltpu.SemaphoreType.DMA,
            pltpu.SemaphoreType.DMA,
        ],
        compiler_params=pltpu.CompilerParams(
            use_tc_tiling_on_sc=False, needs_layout_passes=False),
    )
    mel_pb = jnp.broadcast_to(enc_pos[:seq, :, None], (seq, _D, 16))
    chd_pb = jnp.broadcast_to(dec_pos[:seq, :, None], (seq, _D, 16))
    mel_z, chd_z = run(mel_t, chd_t, melody_emb, chord_emb, mel_pb, chd_pb)
    mel_out = mel_z.transpose(2, 4, 0, 1, 3).reshape(batch, seq, _D)
    chd_out = chd_z.transpose(2, 4, 0, 1, 3).reshape(batch, seq, _D)
    return (mel_out, chd_out)


# speedup vs baseline: 4.0376x; 1.0147x over previous
import jax
import jax.numpy as jnp
from jax import lax
from jax.experimental import pallas as pl
from jax.experimental.pallas import tpu as pltpu
from jax.experimental.pallas import tpu_sc as plsc

_D = 32
_BB = 128
_NW = 32
_KS = 5


def _sc_body(mel_tok, chd_tok, mel_emb, chd_emb, mel_pos, chd_pos,
             mel_z, chd_z, idx_all, pb_v0, pb_v1, buf_v0, buf_v1,
             out_v0, out_v1, gsem0, gsem1, osem0, osem1):
    cid = lax.axis_index("c")
    sid = lax.axis_index("s")
    wid = sid * 2 + cid
    seq = mel_pos.shape[0]
    n_blocks = seq // _KS
    bufs = (buf_v0, buf_v1)
    pbufs = (pb_v0, pb_v1)
    outs = (out_v0, out_v1)
    gsems = (gsem0, gsem1)
    osems = (osem0, osem1)
    iota16 = lax.iota(jnp.int32, 16)
    rows16 = [iota16 + (g * 16) for g in range(_BB // 16)]

    def run_table(tok_hbm, table_hbm, pos_hbm, z_hbm):
        pltpu.sync_copy(tok_hbm.at[wid], idx_all)

        def gather(b, half):
            pltpu.async_copy(
                table_hbm.at[idx_all.at[pl.ds(b * (_KS * _BB), _KS * _BB)]],
                bufs[half], gsems[half])
            pltpu.async_copy(pos_hbm.at[pl.ds(b * _KS, _KS)], pbufs[half],
                             gsems[half])

        def drain_out(half):
            pltpu.make_async_copy(outs[half],
                                  z_hbm.at[pl.ds(0, _KS), :, wid],
                                  osems[half]).wait()

        gather(0, 0)

        def block_pair(b2, carry):
            for half in range(2):
                b = 2 * b2 + half
                nxt = b + 1

                @pl.when(nxt < n_blocks)
                def _():
                    gather(nxt, 1 - half)

                pltpu.make_async_copy(
                    table_hbm.at[idx_all.at[pl.ds(b * (_KS * _BB),
                                                  _KS * _BB)]],
                    bufs[half], gsems[half]).wait()
                pltpu.make_async_copy(pos_hbm.at[pl.ds(b * _KS, _KS)],
                                      pbufs[half], gsems[half]).wait()

                @pl.when(b >= 2)
                def _():
                    drain_out(half)

                for j in range(_KS):
                    jbuf = bufs[half].at[pl.ds(j * _BB, _BB)]
                    jfull = jnp.full((16,), j, jnp.int32)

                    def d_body(d, c3, jbuf=jbuf, jfull=jfull, half=half):
                        dvec = (iota16 + d) & (_D - 1)
                        dt_vec = dvec >> 3
                        di_vec = dvec & 7
                        p = plsc.load_gather(pbufs[half],
                                             [jfull, dvec, iota16])
                        vs = [plsc.load_gather(jbuf, [rows16[g], dvec])
                              for g in range(_BB // 16)]
                        for g in range(_BB // 16):
                            plsc.store_scatter(
                                outs[half],
                                [jfull, dt_vec, di_vec, rows16[g]],
                                vs[g] + p)
                        return c3

                    lax.fori_loop(0, _D, d_body, 0)
                pltpu.async_copy(outs[half],
                                 z_hbm.at[pl.ds(b * _KS, _KS), :, wid],
                                 osems[half])
            return carry

        lax.fori_loop(0, n_blocks // 2, block_pair, 0)
        drain_out(0)
        drain_out(1)

    run_table(mel_tok, mel_emb, mel_pos, mel_z)
    run_table(chd_tok, chd_emb, chd_pos, chd_z)


def kernel(melody_tokens, chord_tokens, melody_emb, chord_emb, enc_pos, dec_pos):
    batch, seq = melody_tokens.shape
    nb = batch // _BB

    def to_worker_major(tok):
        t = jnp.transpose(tok.astype(jnp.int32))
        return t.reshape(seq, nb, _BB).transpose(1, 0, 2).reshape(nb, seq * _BB)

    mel_t = to_worker_major(melody_tokens)
    chd_t = to_worker_major(chord_tokens)

    mesh = plsc.VectorSubcoreMesh(
        core_axis_name="c", subcore_axis_name="s", num_cores=2, num_subcores=16
    )
    z_shape = (seq, _D // 8, batch // _BB, 8, _BB)
    run = pl.kernel(
        _sc_body,
        out_type=(
            jax.ShapeDtypeStruct(z_shape, jnp.float32),
            jax.ShapeDtypeStruct(z_shape, jnp.float32),
        ),
        mesh=mesh,
        scratch_types=[
            pltpu.VMEM((seq * _BB,), jnp.int32),
            pltpu.VMEM((_KS, _D, 16), jnp.float32),
            pltpu.VMEM((_KS, _D, 16), jnp.float32),
            pltpu.VMEM((_KS * _BB, _D), jnp.float32),
            pltpu.VMEM((_KS * _BB, _D), jnp.float32),
            pltpu.VMEM((_KS, _D // 8, 8, _BB), jnp.float32),
            pltpu.VMEM((_KS, _D // 8, 8, _BB), jnp.float32),
            pltpu.SemaphoreType.DMA,
            pltpu.SemaphoreType.DMA,
            pltpu.SemaphoreType.DMA,
            pltpu.SemaphoreType.DMA,
        ],
        compiler_params=pltpu.CompilerParams(
            use_tc_tiling_on_sc=False, needs_layout_passes=False),
    )
    mel_pb = jnp.broadcast_to(enc_pos[:seq, :, None], (seq, _D, 16))
    chd_pb = jnp.broadcast_to(dec_pos[:seq, :, None], (seq, _D, 16))
    mel_z, chd_z = run(mel_t, chd_t, melody_emb, chord_emb, mel_pb, chd_pb)
    mel_out = mel_z.transpose(2, 4, 0, 1, 3).reshape(batch, seq, _D)
    chd_out = chd_z.transpose(2, 4, 0, 1, 3).reshape(batch, seq, _D)
    return (mel_out, chd_out)


# speedup vs baseline: 4.0577x; 1.0050x over previous
import jax
import jax.numpy as jnp
from jax import lax
from jax.experimental import pallas as pl
from jax.experimental.pallas import tpu as pltpu
from jax.experimental.pallas import tpu_sc as plsc

_D = 32
_BB = 128
_KS = 5


def _sc_body(mel_tok, chd_tok, mel_emb, chd_emb, mel_pos, chd_pos,
             mel_z, chd_z, idx_all, pb_v0, pb_v1, buf_v0, buf_v1,
             out_v0, out_v1, gsem0, gsem1, osem0, osem1):
    cid = lax.axis_index("c")
    sid = lax.axis_index("s")
    wid = sid * 2 + cid
    seq = mel_pos.shape[0]
    n_blocks = seq // _KS
    bufs = (buf_v0, buf_v1)
    pbufs = (pb_v0, pb_v1)
    outs = (out_v0, out_v1)
    gsems = (gsem0, gsem1)
    osems = (osem0, osem1)
    iota16 = lax.iota(jnp.int32, 16)
    rows16 = [iota16 + (g * 16) for g in range(_BB // 16)]

    def run_table(tok_hbm, table_hbm, pos_hbm, z_hbm):
        pltpu.sync_copy(tok_hbm.at[wid], idx_all)

        def gather(b, half):
            pltpu.async_copy(
                table_hbm.at[idx_all.at[pl.ds(b * (_KS * _BB), _KS * _BB)]],
                bufs[half], gsems[half])
            pltpu.async_copy(pos_hbm.at[pl.ds(b * _KS, _KS)], pbufs[half],
                             gsems[half])

        def drain_out(half):
            pltpu.make_async_copy(outs[half],
                                  z_hbm.at[pl.ds(0, _KS), :, wid],
                                  osems[half]).wait()

        gather(0, 0)

        def block_pair(b2, carry):
            for half in range(2):
                b = 2 * b2 + half
                nxt = b + 1

                @pl.when(nxt < n_blocks)
                def _():
                    gather(nxt, 1 - half)

                pltpu.make_async_copy(
                    table_hbm.at[idx_all.at[pl.ds(b * (_KS * _BB),
                                                  _KS * _BB)]],
                    bufs[half], gsems[half]).wait()
                pltpu.make_async_copy(pos_hbm.at[pl.ds(b * _KS, _KS)],
                                      pbufs[half], gsems[half]).wait()

                @pl.when(b >= 2)
                def _():
                    drain_out(half)

                for j in range(_KS):
                    jbuf = bufs[half].at[pl.ds(j * _BB, _BB)]
                    jfull = jnp.full((16,), j, jnp.int32)

                    def d_body(d, c3, jbuf=jbuf, jfull=jfull, half=half):
                        dvec = (iota16 + d) & (_D - 1)
                        dt_vec = dvec >> 3
                        di_vec = dvec & 7
                        p = plsc.load_gather(pbufs[half],
                                             [jfull, dvec, iota16])
                        vs = [plsc.load_gather(jbuf, [rows16[g], dvec])
                              for g in range(_BB // 16)]
                        for g in range(_BB // 16):
                            plsc.store_scatter(
                                outs[half],
                                [jfull, dt_vec, di_vec, rows16[g]],
                                vs[g] + p)
                        return c3

                    lax.fori_loop(0, _D, d_body, 0)
                pltpu.async_copy(outs[half],
                                 z_hbm.at[pl.ds(b * _KS, _KS), :, wid],
                                 osems[half])
            return carry

        lax.fori_loop(0, n_blocks // 2, block_pair, 0)
        drain_out(0)
        drain_out(1)

    run_table(mel_tok, mel_emb, mel_pos, mel_z)
    run_table(chd_tok, chd_emb, chd_pos, chd_z)


def kernel(melody_tokens, chord_tokens, melody_emb, chord_emb, enc_pos, dec_pos):
    batch, seq = melody_tokens.shape
    nb = batch // _BB

    def to_worker_major(tok):
        t = jnp.transpose(tok.astype(jnp.int32))
        return t.reshape(seq, nb, _BB).transpose(1, 0, 2).reshape(nb, seq * _BB)

    mel_t = to_worker_major(melody_tokens)
    chd_t = to_worker_major(chord_tokens)

    mesh = plsc.VectorSubcoreMesh(
        core_axis_name="c", subcore_axis_name="s", num_cores=2, num_subcores=16
    )
    z_shape = (seq, _D // 8, batch // _BB, 8, _BB)
    run = pl.kernel(
        _sc_body,
        out_type=(
            jax.ShapeDtypeStruct(z_shape, jnp.float32),
            jax.ShapeDtypeStruct(z_shape, jnp.float32),
        ),
        mesh=mesh,
        scratch_types=[
            pltpu.VMEM((seq * _BB,), jnp.int32),
            pltpu.VMEM((_KS, _D, 16), jnp.float32),
            pltpu.VMEM((_KS, _D, 16), jnp.float32),
            pltpu.VMEM((_KS * _BB, _D), jnp.float32),
            pltpu.VMEM((_KS * _BB, _D), jnp.float32),
            pltpu.VMEM((_KS, _D // 8, 8, _BB), jnp.float32),
            pltpu.VMEM((_KS, _D // 8, 8, _BB), jnp.float32),
            pltpu.SemaphoreType.DMA,
            pltpu.SemaphoreType.DMA,
            pltpu.SemaphoreType.DMA,
            pltpu.SemaphoreType.DMA,
        ],
        compiler_params=pltpu.CompilerParams(
            use_tc_tiling_on_sc=False, needs_layout_passes=False),
    )
    mel_pb = jnp.broadcast_to(enc_pos[:seq, :, None], (seq, _D, 16))
    chd_pb = jnp.broadcast_to(dec_pos[:seq, :, None], (seq, _D, 16))
    mel_z, chd_z = run(mel_t, chd_t, melody_emb, chord_emb, mel_pb, chd_pb)
    mel_out = mel_z.transpose(2, 4, 0, 1, 3).reshape(batch, seq, _D)
    chd_out = chd_z.transpose(2, 4, 0, 1, 3).reshape(batch, seq, _D)
    return (mel_out, chd_out)
